# baseline (speedup 1.0000x reference)
import functools

import jax
import jax.numpy as jnp
from jax import lax
from jax.experimental import pallas as pl
from jax.experimental.pallas import tpu as pltpu
from jax.experimental.pallas import tpu_sc as plsc

_NCLS = 256
_NBINS = _NCLS * _NCLS
_P = 4096


def _mlp_body(x0_ref, x1_ref, x2_ref, seg_ref, w1t_ref, b1_ref, w2t_ref,
              b2_ref, out_ref):
    x0 = x0_ref[:]
    x1 = x1_ref[:]
    x2 = x2_ref[:]
    h = (x0[:, None] * w1t_ref[0, :][None, :]
         + x1[:, None] * w1t_ref[1, :][None, :]
         + x2[:, None] * w1t_ref[2, :][None, :]
         + b1_ref[:][None, :])
    h = jnp.maximum(h, 0.0)
    logits = jnp.dot(h, w2t_ref[:], preferred_element_type=jnp.float32)
    logits = logits + b2_ref[:][None, :]
    m = jnp.max(logits, axis=1, keepdims=True)
    iota = lax.broadcasted_iota(jnp.int32, logits.shape, 1)
    pred = jnp.min(jnp.where(logits == m, iota, _NBINS), axis=1)
    gt = (seg_ref[:] * 255.0).astype(jnp.int32)
    out_ref[:] = pred * _NCLS + gt


def _merge_indices(x0, x1, x2, seg, w1t, b1, w2t, b2):
    n = x0.shape[0]
    grid = (n // _P,)
    return pl.pallas_call(
        _mlp_body,
        grid=grid,
        in_specs=[
            pl.BlockSpec((_P,), lambda i: (i,)),
            pl.BlockSpec((_P,), lambda i: (i,)),
            pl.BlockSpec((_P,), lambda i: (i,)),
            pl.BlockSpec((_P,), lambda i: (i,)),
            pl.BlockSpec((3, 128), lambda i: (0, 0)),
            pl.BlockSpec((128,), lambda i: (0,)),
            pl.BlockSpec((128, _NCLS), lambda i: (0, 0)),
            pl.BlockSpec((_NCLS,), lambda i: (0,)),
        ],
        out_specs=pl.BlockSpec((_P,), lambda i: (i,)),
        out_shape=jax.ShapeDtypeStruct((n,), jnp.int32),
    )(x0, x1, x2, seg, w1t, b1, w2t, b2)


def _hist_body(merge_hbm, partial_hbm, out_hbm, tab_v, chunk_v, acc_v,
               stage_v, chunk, n_img):
    c = lax.axis_index("c")
    s = lax.axis_index("s")
    row = c * 16 + s
    zeros16 = jnp.zeros((16,), jnp.int32)
    ones16 = jnp.ones((16,), jnp.int32)

    def zbody(i, _):
        tab_v[pl.ds(i * 16, 16)] = zeros16
        return 0

    lax.fori_loop(0, _NBINS // 16, zbody, 0, unroll=8)

    pltpu.sync_copy(merge_hbm.at[pl.ds(row * chunk, chunk)], chunk_v)

    def sbody(i, _):
        idx = chunk_v[pl.ds(i * 16, 16)]
        plsc.addupdate_scatter(tab_v, [idx], ones16)
        return 0

    lax.fori_loop(0, chunk // 16, sbody, 0, unroll=4)

    pltpu.sync_copy(tab_v, partial_hbm.at[row])
    plsc.subcore_barrier()

    sl = _NBINS // 16
    base = s * sl

    def zbody2(i, _):
        acc_v[pl.ds(i * 16, 16)] = zeros16
        return 0

    lax.fori_loop(0, sl // 16, zbody2, 0, unroll=8)

    def rbody(j, _):
        pltpu.sync_copy(partial_hbm.at[c * 16 + j, pl.ds(base, sl)], stage_v)

        def abody(k, _):
            acc_v[pl.ds(k * 16, 16)] = (acc_v[pl.ds(k * 16, 16)]
                                        + stage_v[pl.ds(k * 16, 16)])
            return 0

        lax.fori_loop(0, sl // 16, abody, 0, unroll=8)
        return 0

    lax.fori_loop(0, 16, rbody, 0)
    pltpu.sync_copy(acc_v, out_hbm.at[c, pl.ds(base, sl)])


def _histogram(merge, n_img):
    n = merge.shape[0]
    chunk = n // 32
    mesh = plsc.VectorSubcoreMesh(core_axis_name="c", subcore_axis_name="s")
    body = functools.partial(_hist_body, chunk=chunk, n_img=n_img)
    f = pl.kernel(
        body,
        out_type=[
            jax.ShapeDtypeStruct((32, _NBINS), jnp.int32),
            jax.ShapeDtypeStruct((n_img, _NBINS), jnp.int32),
        ],
        mesh=mesh,
        compiler_params=pltpu.CompilerParams(needs_layout_passes=False),
        scratch_types=[
            pltpu.VMEM((_NBINS,), jnp.int32),
            pltpu.VMEM((chunk,), jnp.int32),
            pltpu.VMEM((_NBINS // 16,), jnp.int32),
            pltpu.VMEM((_NBINS // 16,), jnp.int32),
        ],
    )
    _, hist = f(merge)
    return hist


def kernel(fake_images, segmaps, W1, b1, W2, b2):
    B, C, H, W = fake_images.shape
    n = B * H * W
    x0 = fake_images[:, 0].reshape(n)
    x1 = fake_images[:, 1].reshape(n)
    x2 = fake_images[:, 2].reshape(n)
    seg = segmaps.reshape(n)
    merge = _merge_indices(x0, x1, x2, seg, W1.T, b1, W2.T, b2)
    hist = _histogram(merge, B)
    return hist.reshape(B, _NCLS, _NCLS)


# speedup vs baseline: 2.6782x; 2.6782x over previous
import functools

import jax
import jax.numpy as jnp
from jax import lax
from jax.experimental import pallas as pl
from jax.experimental.pallas import tpu as pltpu
from jax.experimental.pallas import tpu_sc as plsc

_NCLS = 256
_NBINS = _NCLS * _NCLS
_P = 4096


def _mlp_body(x_ref, seg_ref, w1_ref, b1_ref, w2_ref, b2_ref, out_ref):
    h = lax.dot_general(w1_ref[:], x_ref[:], (((1,), (0,)), ((), ())),
                        preferred_element_type=jnp.float32)
    h = jnp.maximum(h + b1_ref[:], 0.0)
    logits = lax.dot_general(w2_ref[:], h, (((1,), (0,)), ((), ())),
                             preferred_element_type=jnp.float32)
    logits = logits + b2_ref[:]
    m = jnp.max(logits, axis=0, keepdims=True)
    iota = lax.broadcasted_iota(jnp.int32, logits.shape, 0)
    pred = jnp.min(jnp.where(logits == m, iota, _NBINS), axis=0)
    gt = (seg_ref[:] * 255.0).astype(jnp.int32)
    out_ref[:] = pred * _NCLS + gt


def _merge_indices(x, seg, w1, b1, w2, b2):
    n = seg.shape[0]
    grid = (n // _P,)
    return pl.pallas_call(
        _mlp_body,
        grid=grid,
        in_specs=[
            pl.BlockSpec((3, _P), lambda i: (0, i)),
            pl.BlockSpec((_P,), lambda i: (i,)),
            pl.BlockSpec((128, 3), lambda i: (0, 0)),
            pl.BlockSpec((128, 1), lambda i: (0, 0)),
            pl.BlockSpec((_NCLS, 128), lambda i: (0, 0)),
            pl.BlockSpec((_NCLS, 1), lambda i: (0, 0)),
        ],
        out_specs=pl.BlockSpec((_P,), lambda i: (i,)),
        out_shape=jax.ShapeDtypeStruct((n,), jnp.int32),
    )(x, seg, w1, b1, w2, b2)


def _hist_body(merge_hbm, out_hbm, tab_v, chunk_v, acc_v, stage_v, shared,
               chunk, n_img):
    c = lax.axis_index("c")
    s = lax.axis_index("s")
    row = c * 16 + s
    zeros16 = jnp.zeros((16,), jnp.int32)
    ones16 = jnp.ones((16,), jnp.int32)

    def zbody(i, _):
        tab_v[pl.ds(i * 16, 16)] = zeros16
        return 0

    lax.fori_loop(0, _NBINS // 16, zbody, 0, unroll=8)

    pltpu.sync_copy(merge_hbm.at[pl.ds(row * chunk, chunk)], chunk_v)

    def sbody(i, _):
        idx = chunk_v[pl.ds(i * 16, 16)]
        plsc.addupdate_scatter(tab_v, [idx], ones16)
        return 0

    lax.fori_loop(0, chunk // 16, sbody, 0, unroll=4)

    half = _NBINS // 2
    sl = half // 16
    base = s * sl
    for h in range(2):
        pltpu.sync_copy(tab_v.at[pl.ds(h * half, half)], shared.at[s])
        plsc.subcore_barrier()

        def zbody2(i, _):
            acc_v[pl.ds(i * 16, 16)] = zeros16
            return 0

        lax.fori_loop(0, sl // 16, zbody2, 0, unroll=8)

        def rbody(j, _):
            pltpu.sync_copy(shared.at[j, pl.ds(base, sl)], stage_v)

            def abody(k, _):
                acc_v[pl.ds(k * 16, 16)] = (acc_v[pl.ds(k * 16, 16)]
                                            + stage_v[pl.ds(k * 16, 16)])
                return 0

            lax.fori_loop(0, sl // 16, abody, 0, unroll=8)
            return 0

        lax.fori_loop(0, 16, rbody, 0)
        pltpu.sync_copy(acc_v, out_hbm.at[c, pl.ds(h * half + base, sl)])
        plsc.subcore_barrier()


def _histogram(merge, n_img):
    n = merge.shape[0]
    chunk = n // 32
    mesh = plsc.VectorSubcoreMesh(core_axis_name="c", subcore_axis_name="s")
    body = functools.partial(_hist_body, chunk=chunk, n_img=n_img)
    f = pl.kernel(
        body,
        out_type=jax.ShapeDtypeStruct((n_img, _NBINS), jnp.int32),
        mesh=mesh,
        compiler_params=pltpu.CompilerParams(needs_layout_passes=False),
        scratch_types=[
            pltpu.VMEM((_NBINS,), jnp.int32),
            pltpu.VMEM((chunk,), jnp.int32),
            pltpu.VMEM((_NBINS // 32,), jnp.int32),
            pltpu.VMEM((_NBINS // 32,), jnp.int32),
            pltpu.VMEM_SHARED((16, _NBINS // 2), jnp.int32),
        ],
    )
    return f(merge)


def kernel(fake_images, segmaps, W1, b1, W2, b2):
    B, C, H, W = fake_images.shape
    n = B * H * W
    x = jnp.moveaxis(fake_images.reshape(B, C, H * W), 1, 0).reshape(C, n)
    seg = segmaps.reshape(n)
    merge = _merge_indices(x, seg, W1, b1[:, None], W2, b2[:, None])
    hist = _histogram(merge, B)
    return hist.reshape(B, _NCLS, _NCLS)


# speedup vs baseline: 2.9368x; 1.0966x over previous
import functools

import jax
import jax.numpy as jnp
from jax import lax
from jax.experimental import pallas as pl
from jax.experimental.pallas import tpu as pltpu
from jax.experimental.pallas import tpu_sc as plsc

_NCLS = 256
_NBINS = _NCLS * _NCLS
_P = 8192


def _mlp_body(x_ref, seg_ref, w1_ref, b1_ref, w2_ref, b2_ref, out_ref):
    h = lax.dot_general(w1_ref[:], x_ref[:], (((1,), (0,)), ((), ())),
                        preferred_element_type=jnp.float32)
    h = jnp.maximum(h + b1_ref[:], 0.0)
    logits = lax.dot_general(w2_ref[:], h, (((1,), (0,)), ((), ())),
                             preferred_element_type=jnp.float32)
    logits = logits + b2_ref[:]
    m = jnp.max(logits, axis=0, keepdims=True)
    iota = lax.broadcasted_iota(jnp.int32, logits.shape,
                                0).astype(jnp.float32)
    pred = jnp.min(jnp.where(logits == m, iota, float(_NBINS)),
                   axis=0).astype(jnp.int32)
    gt = (seg_ref[:] * 255.0).astype(jnp.int32)
    out_ref[:] = pred * _NCLS + gt


def _merge_indices(x, seg, w1, b1, w2, b2):
    n = seg.shape[0]
    grid = (n // _P,)
    return pl.pallas_call(
        _mlp_body,
        grid=grid,
        in_specs=[
            pl.BlockSpec((3, _P), lambda i: (0, i)),
            pl.BlockSpec((_P,), lambda i: (i,)),
            pl.BlockSpec((128, 3), lambda i: (0, 0)),
            pl.BlockSpec((128, 1), lambda i: (0, 0)),
            pl.BlockSpec((_NCLS, 128), lambda i: (0, 0)),
            pl.BlockSpec((_NCLS, 1), lambda i: (0, 0)),
        ],
        out_specs=pl.BlockSpec((_P,), lambda i: (i,)),
        out_shape=jax.ShapeDtypeStruct((n,), jnp.int32),
    )(x, seg, w1, b1, w2, b2)


def _hist_body(merge_hbm, out_hbm, tab_v, chunk_v, acc_v, stage_v, shared,
               chunk, n_img):
    c = lax.axis_index("c")
    s = lax.axis_index("s")
    row = c * 16 + s
    zeros16 = jnp.zeros((16,), jnp.int32)
    ones16 = jnp.ones((16,), jnp.int32)

    def zbody(i, _):
        tab_v[pl.ds(i * 16, 16)] = zeros16
        return 0

    lax.fori_loop(0, _NBINS // 16, zbody, 0, unroll=8)

    pltpu.sync_copy(merge_hbm.at[pl.ds(row * chunk, chunk)], chunk_v)

    def sbody(i, _):
        idx = chunk_v[pl.ds(i * 16, 16)]
        plsc.addupdate_scatter(tab_v, [idx], ones16)
        return 0

    lax.fori_loop(0, chunk // 16, sbody, 0, unroll=4)

    half = _NBINS // 2
    sl = half // 16
    base = s * sl
    for h in range(2):
        pltpu.sync_copy(tab_v.at[pl.ds(h * half, half)], shared.at[s])
        plsc.subcore_barrier()

        def zbody2(i, _):
            acc_v[pl.ds(i * 16, 16)] = zeros16
            return 0

        lax.fori_loop(0, sl // 16, zbody2, 0, unroll=8)

        def rbody(j, _):
            pltpu.sync_copy(shared.at[j, pl.ds(base, sl)], stage_v)

            def abody(k, _):
                acc_v[pl.ds(k * 16, 16)] = (acc_v[pl.ds(k * 16, 16)]
                                            + stage_v[pl.ds(k * 16, 16)])
                return 0

            lax.fori_loop(0, sl // 16, abody, 0, unroll=8)
            return 0

        lax.fori_loop(0, 16, rbody, 0)
        pltpu.sync_copy(acc_v, out_hbm.at[c, pl.ds(h * half + base, sl)])
        plsc.subcore_barrier()


def _histogram(merge, n_img):
    n = merge.shape[0]
    chunk = n // 32
    mesh = plsc.VectorSubcoreMesh(core_axis_name="c", subcore_axis_name="s")
    body = functools.partial(_hist_body, chunk=chunk, n_img=n_img)
    f = pl.kernel(
        body,
        out_type=jax.ShapeDtypeStruct((n_img, _NBINS), jnp.int32),
        mesh=mesh,
        compiler_params=pltpu.CompilerParams(needs_layout_passes=False),
        scratch_types=[
            pltpu.VMEM((_NBINS,), jnp.int32),
            pltpu.VMEM((chunk,), jnp.int32),
            pltpu.VMEM((_NBINS // 32,), jnp.int32),
            pltpu.VMEM((_NBINS // 32,), jnp.int32),
            pltpu.VMEM_SHARED((16, _NBINS // 2), jnp.int32),
        ],
    )
    return f(merge)


def kernel(fake_images, segmaps, W1, b1, W2, b2):
    B, C, H, W = fake_images.shape
    n = B * H * W
    x = jnp.moveaxis(fake_images.reshape(B, C, H * W), 1, 0).reshape(C, n)
    seg = segmaps.reshape(n)
    merge = _merge_indices(x, seg, W1, b1[:, None], W2, b2[:, None])
    hist = _histogram(merge, B)
    return hist.reshape(B, _NCLS, _NCLS)
